# baseline (speedup 1.0000x reference)
import functools

import jax
import jax.numpy as jnp
from jax.experimental import pallas as pl
from jax.experimental.pallas import tpu as pltpu
from jax.sharding import PartitionSpec as P


_N_DEV = min(2, jax.device_count())
if _N_DEV > 1:
    _MESH = jax.make_mesh(
        (_N_DEV,), ("d",),
        axis_types=(jax.sharding.AxisType.Explicit,),
    )
    jax.sharding.set_mesh(_MESH)
else:
    _MESH = None


def _gcn_block(x_ref, t_ref, w_ref, b_ref, o_ref):
    k = pl.program_id(1)

    @pl.when(k == 0)
    def _init():
        o_ref[...] = jnp.broadcast_to(b_ref[...], o_ref.shape)

    t = t_ref[0].astype(jnp.bfloat16)
    w = w_ref[0]
    n_batch = x_ref.shape[0]
    for b in range(n_batch):
        temp = jnp.dot(t, x_ref[b], preferred_element_type=jnp.float32)
        part = jnp.dot(temp.astype(jnp.bfloat16), w,
                       preferred_element_type=jnp.float32)
        o_ref[b] = o_ref[b] + part


def _local_gcn(input, T_loc, weight, bias):
    B, V, D_IN = input.shape
    K, V_loc, _ = T_loc.shape
    D_OUT = weight.shape[-1]
    TILE_V = min(512, V_loc)

    x16 = input.astype(jnp.bfloat16)
    w16 = weight.astype(jnp.bfloat16)
    bias2d = bias.reshape(1, D_OUT)

    return pl.pallas_call(
        _gcn_block,
        grid=(V_loc // TILE_V, K),
        in_specs=[
            pl.BlockSpec((B, V, D_IN), lambda i, k: (0, 0, 0)),
            pl.BlockSpec((1, TILE_V, V), lambda i, k: (k, i, 0)),
            pl.BlockSpec((1, D_IN, D_OUT), lambda i, k: (k, 0, 0)),
            pl.BlockSpec((1, D_OUT), lambda i, k: (0, 0)),
        ],
        out_specs=pl.BlockSpec((B, TILE_V, D_OUT), lambda i, k: (0, i, 0)),
        out_shape=jax.ShapeDtypeStruct((B, V_loc, D_OUT), jnp.float32),
        compiler_params=pltpu.CompilerParams(
            vmem_limit_bytes=60 * 1024 * 1024,
        ),
    )(x16, T_loc, w16, bias2d)


def _kernel_impl(input, T_k, weight, bias):
    if _MESH is not None and T_k.shape[1] % _N_DEV == 0:
        input = jax.reshard(input, P())
        T_k = jax.reshard(T_k, P(None, "d", None))
        weight = jax.reshard(weight, P())
        bias = jax.reshard(bias, P())
        fn = jax.shard_map(
            _local_gcn,
            mesh=_MESH,
            in_specs=(P(), P(None, "d", None), P(), P()),
            out_specs=P(None, "d", None),
            check_vma=False,
        )
        return fn(input, T_k, weight, bias)
    return _local_gcn(input, T_k, weight, bias)


if _MESH is not None:
    _NS = lambda spec: jax.sharding.NamedSharding(_MESH, spec)
    kernel = jax.jit(
        _kernel_impl,
        in_shardings=(_NS(P()), _NS(P(None, "d", None)), _NS(P()), _NS(P())),
        out_shardings=_NS(P(None, "d", None)),
    )
else:
    kernel = jax.jit(_kernel_impl)


# speedup vs baseline: 1.0742x; 1.0742x over previous
import functools

import jax
import jax.numpy as jnp
from jax.experimental import pallas as pl
from jax.experimental.pallas import tpu as pltpu
from jax.sharding import PartitionSpec as P


_N_DEV = min(2, jax.device_count())
if _N_DEV > 1:
    _MESH = jax.make_mesh(
        (_N_DEV,), ("d",),
        axis_types=(jax.sharding.AxisType.Explicit,),
    )
    jax.sharding.set_mesh(_MESH)
else:
    _MESH = None


def _gcn_block(x_ref, t_ref, w_ref, b_ref, o_ref):
    k = pl.program_id(1)

    @pl.when(k == 0)
    def _init():
        o_ref[...] = jnp.broadcast_to(b_ref[...], o_ref.shape)

    t = t_ref[0].astype(jnp.bfloat16)
    w = w_ref[0]
    n_batch = x_ref.shape[0]
    temps = [
        jnp.dot(t, x_ref[b], preferred_element_type=jnp.float32).astype(
            jnp.bfloat16)
        for b in range(n_batch)
    ]
    for b in range(n_batch):
        part = jnp.dot(temps[b], w, preferred_element_type=jnp.float32)
        o_ref[b] = o_ref[b] + part


def _local_gcn(input, T_loc, weight, bias):
    B, V, D_IN = input.shape
    K, V_loc, _ = T_loc.shape
    D_OUT = weight.shape[-1]
    TILE_V = min(512, V_loc)

    x16 = input.astype(jnp.bfloat16)
    w16 = weight.astype(jnp.bfloat16)
    bias2d = bias.reshape(1, D_OUT)

    return pl.pallas_call(
        _gcn_block,
        grid=(V_loc // TILE_V, K),
        in_specs=[
            pl.BlockSpec((B, V, D_IN), lambda i, k: (0, 0, 0)),
            pl.BlockSpec((1, TILE_V, V), lambda i, k: (k, i, 0)),
            pl.BlockSpec((1, D_IN, D_OUT), lambda i, k: (k, 0, 0)),
            pl.BlockSpec((1, D_OUT), lambda i, k: (0, 0)),
        ],
        out_specs=pl.BlockSpec((B, TILE_V, D_OUT), lambda i, k: (0, i, 0)),
        out_shape=jax.ShapeDtypeStruct((B, V_loc, D_OUT), jnp.float32),
        compiler_params=pltpu.CompilerParams(
            vmem_limit_bytes=60 * 1024 * 1024,
        ),
    )(x16, T_loc, w16, bias2d)


def _kernel_impl(input, T_k, weight, bias):
    if _MESH is not None and T_k.shape[1] % _N_DEV == 0:
        input = jax.reshard(input, P())
        T_k = jax.reshard(T_k, P(None, "d", None))
        weight = jax.reshard(weight, P())
        bias = jax.reshard(bias, P())
        fn = jax.shard_map(
            _local_gcn,
            mesh=_MESH,
            in_specs=(P(), P(None, "d", None), P(), P()),
            out_specs=P(None, "d", None),
            check_vma=False,
        )
        return fn(input, T_k, weight, bias)
    return _local_gcn(input, T_k, weight, bias)


if _MESH is not None:
    _NS = lambda spec: jax.sharding.NamedSharding(_MESH, spec)
    kernel = jax.jit(
        _kernel_impl,
        in_shardings=(_NS(P()), _NS(P(None, "d", None)), _NS(P()), _NS(P())),
        out_shardings=_NS(P(None, "d", None)),
    )
else:
    kernel = jax.jit(_kernel_impl)


# speedup vs baseline: 1.1551x; 1.0753x over previous
import functools

import jax
import jax.numpy as jnp
from jax.experimental import pallas as pl
from jax.experimental.pallas import tpu as pltpu
from jax.sharding import PartitionSpec as P


_N_DEV = min(2, jax.device_count())
if _N_DEV > 1:
    _MESH = jax.make_mesh(
        (_N_DEV,), ("d",),
        axis_types=(jax.sharding.AxisType.Explicit,),
    )
    jax.sharding.set_mesh(_MESH)
else:
    _MESH = None


def _gcn_block(x_ref, t_ref, w_ref, b_ref, o_ref):
    k = pl.program_id(1)

    @pl.when(k == 0)
    def _init():
        o_ref[...] = jnp.broadcast_to(b_ref[...], o_ref.shape)

    t = t_ref[0]
    w = w_ref[0]
    n_batch = x_ref.shape[0]
    for b in range(n_batch):
        temp = jnp.dot(t, x_ref[b], preferred_element_type=jnp.float32)
        part = jnp.dot(temp, w, preferred_element_type=jnp.float32)
        o_ref[b] = o_ref[b] + part


def _local_gcn(input, T_loc, weight, bias):
    B, V, D_IN = input.shape
    K, V_loc, _ = T_loc.shape
    D_OUT = weight.shape[-1]
    TILE_V = min(512, V_loc)

    bias2d = bias.reshape(1, D_OUT)

    return pl.pallas_call(
        _gcn_block,
        grid=(V_loc // TILE_V, K),
        in_specs=[
            pl.BlockSpec((B, V, D_IN), lambda i, k: (0, 0, 0)),
            pl.BlockSpec((1, TILE_V, V), lambda i, k: (k, i, 0)),
            pl.BlockSpec((1, D_IN, D_OUT), lambda i, k: (k, 0, 0)),
            pl.BlockSpec((1, D_OUT), lambda i, k: (0, 0)),
        ],
        out_specs=pl.BlockSpec((B, TILE_V, D_OUT), lambda i, k: (0, i, 0)),
        out_shape=jax.ShapeDtypeStruct((B, V_loc, D_OUT), jnp.float32),
        compiler_params=pltpu.CompilerParams(
            vmem_limit_bytes=63 * 1024 * 1024,
        ),
    )(input, T_loc, weight, bias2d)


def _kernel_impl(input, T_k, weight, bias):
    if _MESH is not None and T_k.shape[1] % _N_DEV == 0:
        input = jax.reshard(input, P())
        T_k = jax.reshard(T_k, P(None, "d", None))
        weight = jax.reshard(weight, P())
        bias = jax.reshard(bias, P())
        fn = jax.shard_map(
            _local_gcn,
            mesh=_MESH,
            in_specs=(P(), P(None, "d", None), P(), P()),
            out_specs=P(None, "d", None),
            check_vma=False,
        )
        return fn(input, T_k, weight, bias)
    return _local_gcn(input, T_k, weight, bias)


if _MESH is not None:
    _NS = lambda spec: jax.sharding.NamedSharding(_MESH, spec)
    kernel = jax.jit(
        _kernel_impl,
        in_shardings=(_NS(P()), _NS(P(None, "d", None)), _NS(P()), _NS(P())),
        out_shardings=_NS(P(None, "d", None)),
    )
else:
    kernel = jax.jit(_kernel_impl)


# speedup vs baseline: 1.7059x; 1.4768x over previous
import functools

import jax
import jax.numpy as jnp
from jax.experimental import pallas as pl
from jax.experimental.pallas import tpu as pltpu
from jax.sharding import PartitionSpec as P


_N_DEV = min(2, jax.device_count())
if _N_DEV > 1:
    _MESH = jax.make_mesh(
        (_N_DEV,), ("d",),
        axis_types=(jax.sharding.AxisType.Explicit,),
    )
    jax.sharding.set_mesh(_MESH)
else:
    _MESH = None


def _gcn_block(off_ref, x_ref, t_ref, w_ref, b_ref, o_ref):
    k = pl.program_id(1)

    @pl.when(k == 0)
    def _init():
        o_ref[...] = jnp.broadcast_to(b_ref[...], o_ref.shape)

    t = t_ref[0]
    w = w_ref[0]
    n_batch = x_ref.shape[0]
    for b in range(n_batch):
        temp = jnp.dot(t, x_ref[b], preferred_element_type=jnp.float32)
        part = jnp.dot(temp, w, preferred_element_type=jnp.float32)
        o_ref[b] = o_ref[b] + part


def _sharded_gcn(input, T_k, weight, bias, n_shards):
    B, V, D_IN = input.shape
    K = T_k.shape[0]
    D_OUT = weight.shape[-1]
    V_loc = V // n_shards
    TILE_V = min(512, V_loc)
    n_i = V_loc // TILE_V

    bias2d = bias.reshape(1, D_OUT)
    off = (jax.lax.axis_index("d") * n_i).reshape(1).astype(jnp.int32)

    return pl.pallas_call(
        _gcn_block,
        grid_spec=pltpu.PrefetchScalarGridSpec(
            num_scalar_prefetch=1,
            grid=(n_i, K),
            in_specs=[
                pl.BlockSpec((B, V, D_IN), lambda i, k, off: (0, 0, 0)),
                pl.BlockSpec((1, TILE_V, V), lambda i, k, off: (k, off[0] + i, 0)),
                pl.BlockSpec((1, D_IN, D_OUT), lambda i, k, off: (k, 0, 0)),
                pl.BlockSpec((1, D_OUT), lambda i, k, off: (0, 0)),
            ],
            out_specs=pl.BlockSpec((B, TILE_V, D_OUT), lambda i, k, off: (0, i, 0)),
        ),
        out_shape=jax.ShapeDtypeStruct((B, V_loc, D_OUT), jnp.float32),
        compiler_params=pltpu.CompilerParams(
            vmem_limit_bytes=63 * 1024 * 1024,
        ),
    )(off, input, T_k, weight, bias2d)


def _local_gcn(input, T_loc, weight, bias):
    B, V, D_IN = input.shape
    K, V_loc, _ = T_loc.shape
    D_OUT = weight.shape[-1]
    TILE_V = min(512, V_loc)

    bias2d = bias.reshape(1, D_OUT)

    return pl.pallas_call(
        lambda x_ref, t_ref, w_ref, b_ref, o_ref: _gcn_block(
            None, x_ref, t_ref, w_ref, b_ref, o_ref),
        grid=(V_loc // TILE_V, K),
        in_specs=[
            pl.BlockSpec((B, V, D_IN), lambda i, k: (0, 0, 0)),
            pl.BlockSpec((1, TILE_V, V), lambda i, k: (k, i, 0)),
            pl.BlockSpec((1, D_IN, D_OUT), lambda i, k: (k, 0, 0)),
            pl.BlockSpec((1, D_OUT), lambda i, k: (0, 0)),
        ],
        out_specs=pl.BlockSpec((B, TILE_V, D_OUT), lambda i, k: (0, i, 0)),
        out_shape=jax.ShapeDtypeStruct((B, V_loc, D_OUT), jnp.float32),
        compiler_params=pltpu.CompilerParams(
            vmem_limit_bytes=63 * 1024 * 1024,
        ),
    )(input, T_loc, weight, bias2d)


def _kernel_impl(input, T_k, weight, bias):
    if _MESH is not None and T_k.shape[1] % _N_DEV == 0:
        input = jax.reshard(input, P())
        T_k = jax.reshard(T_k, P())
        weight = jax.reshard(weight, P())
        bias = jax.reshard(bias, P())
        fn = jax.shard_map(
            lambda x, t, w, b: _sharded_gcn(x, t, w, b, _N_DEV),
            mesh=_MESH,
            in_specs=(P(), P(), P(), P()),
            out_specs=P(None, "d", None),
            check_vma=False,
        )
        return fn(input, T_k, weight, bias)
    return _local_gcn(input, T_k, weight, bias)


if _MESH is not None:
    _NS = lambda spec: jax.sharding.NamedSharding(_MESH, spec)
    kernel = jax.jit(
        _kernel_impl,
        in_shardings=(_NS(P()), _NS(P()), _NS(P()), _NS(P())),
        out_shardings=_NS(P(None, "d", None)),
    )
else:
    kernel = jax.jit(_kernel_impl)
